# baseline (speedup 1.0000x reference)
import jax
import jax.numpy as jnp
from jax.experimental import pallas as pl
from jax.experimental.pallas import tpu as pltpu

NUM_EXPERTS = 8
TOP_K = 2
BLK = 2048
N_TOKENS = 32768
HALF_STEPS = (N_TOKENS // 2) // BLK


def _half(x, wt, b):
    logits = jnp.dot(x, wt, preferred_element_type=jnp.float32) + b
    lt = logits.T
    m = jnp.max(lt, axis=0, keepdims=True)
    e = jnp.exp(lt - m)
    s = jnp.sum(e, axis=0, keepdims=True)
    pt = e / s

    eids = jax.lax.broadcasted_iota(jnp.int32, pt.shape, 0)
    p1 = jnp.max(pt, axis=0, keepdims=True)
    i1 = jnp.min(jnp.where(pt == p1, eids, NUM_EXPERTS), axis=0,
                 keepdims=True)
    pt2 = jnp.where(eids == i1, -1.0, pt)
    p2 = jnp.max(pt2, axis=0, keepdims=True)
    i2 = jnp.min(jnp.where(pt2 == p2, eids, NUM_EXPERTS), axis=0,
                 keepdims=True)

    t = jnp.exp(p2 - p1)
    denom = 1.0 + t
    wv = jnp.concatenate([1.0 / denom, t / denom], axis=0)
    iv = jnp.concatenate([i1, i2], axis=0)
    return lt, pt, wv, iv


def _router_body(xa_ref, xb_ref, wt_ref, b_ref, *out_refs):
    wt_v = wt_ref[...]
    b_v = b_ref[...]
    la, pa, wa, ia = _half(xa_ref[...], wt_v, b_v)
    lb, pb, wb, ib = _half(xb_ref[...], wt_v, b_v)
    (la_ref, pa_ref, wa_ref, ia_ref,
     lb_ref, pb_ref, wb_ref, ib_ref) = out_refs
    la_ref[...] = la
    pa_ref[...] = pa
    wa_ref[...] = wa
    ia_ref[...] = ia
    lb_ref[...] = lb
    pb_ref[...] = pb
    wb_ref[...] = wb
    ib_ref[...] = ib


@jax.jit
def kernel(inputs, W, b):
    B, S, D = inputs.shape
    N = B * S
    H = N // 2
    x2d = inputs.reshape(N, D)
    wt = W.T
    b2d = b.reshape(1, NUM_EXPERTS)

    grid = (HALF_STEPS,)
    o_e = pl.BlockSpec((NUM_EXPERTS, BLK), lambda i: (0, i))
    o_k = pl.BlockSpec((TOP_K, BLK), lambda i: (0, i))
    sh_e = jax.ShapeDtypeStruct((NUM_EXPERTS, H), jnp.float32)
    sh_k = jax.ShapeDtypeStruct((TOP_K, H), jnp.float32)
    sh_ki = jax.ShapeDtypeStruct((TOP_K, H), jnp.int32)

    outs = pl.pallas_call(
        _router_body,
        grid=grid,
        in_specs=[
            pl.BlockSpec((BLK, D), lambda i: (i, 0)),
            pl.BlockSpec((BLK, D), lambda i: (i + HALF_STEPS, 0)),
            pl.BlockSpec((D, NUM_EXPERTS), lambda i: (0, 0)),
            pl.BlockSpec((1, NUM_EXPERTS), lambda i: (0, 0)),
        ],
        out_specs=[o_e, o_e, o_k, o_k, o_e, o_e, o_k, o_k],
        out_shape=[sh_e, sh_e, sh_k, sh_ki, sh_e, sh_e, sh_k, sh_ki],
    )(x2d, x2d, wt, b2d)
    la, pa, wa, ia, lb, pb, wb, ib = outs

    logits_t = jnp.concatenate([la, lb], axis=1)
    probs_t = jnp.concatenate([pa, pb], axis=1)
    w_t = jnp.concatenate([wa, wb], axis=1)
    idx_t = jnp.concatenate([ia, ib], axis=1)
    return (
        logits_t.T.reshape(B, S, NUM_EXPERTS),
        probs_t.T.reshape(B, S, NUM_EXPERTS),
        w_t.T.reshape(B, S, TOP_K),
        idx_t.T.reshape(B, S, TOP_K),
    )
